# baseline (speedup 1.0000x reference)
import functools

import jax
import jax.numpy as jnp
import numpy as np
from jax import lax
from jax.experimental import pallas as pl
from jax.experimental.pallas import tpu as pltpu
from jax.experimental.pallas import tpu_sc as plsc

_B = 16
_N = 2048
_D = 512

_NC = 2
_NS = 16
_NW = _NC * _NS

_ROWS = _B * _N
_ROWS_PER_W = _ROWS // _NW
_CHUNK = 64
_NCHUNK = _ROWS_PER_W // _CHUNK


def _threefry2x32(k1, k2, x1, x2):
    rot = ([13, 15, 26, 6], [17, 29, 16, 24])
    with np.errstate(over="ignore"):
        ks = [k1, k2, np.uint32(k1 ^ k2 ^ np.uint32(0x1BD11BDA))]
        x = [(x1 + ks[0]).astype(np.uint32), (x2 + ks[1]).astype(np.uint32)]
        for d in range(5):
            for r in rot[d % 2]:
                x[0] = (x[0] + x[1]).astype(np.uint32)
                x[1] = ((x[1] << np.uint32(r))
                        | (x[1] >> np.uint32(32 - r))).astype(np.uint32)
                x[1] = x[0] ^ x[1]
            x[0] = (x[0] + ks[(d + 1) % 3]).astype(np.uint32)
            x[1] = (x[1] + ks[(d + 2) % 3] + np.uint32(d + 1)).astype(np.uint32)
    return x[0], x[1]


def _compute_perm(seed: int, n: int) -> np.ndarray:
    key = (np.uint32(seed >> 32), np.uint32(seed & 0xFFFFFFFF))
    x = np.arange(n)
    num_rounds = int(np.ceil(3 * np.log(max(1, n))
                             / np.log(np.iinfo(np.uint32).max)))
    for _ in range(num_rounds):
        b1, b2 = _threefry2x32(*key, np.zeros(2, np.uint32),
                               np.arange(2, dtype=np.uint32))
        key = (b1[0], b2[0])
        s1, s2 = _threefry2x32(b1[1], b2[1], np.zeros(n, np.uint32),
                               np.arange(n, dtype=np.uint32))
        x = x[np.argsort(s1 ^ s2, kind="stable")]
    return x


_PERM = _compute_perm(1234, _N)
_INV_PERM = np.argsort(_PERM)
_DST = np.ascontiguousarray(
    (_INV_PERM[None, :] + np.arange(_B, dtype=np.int64)[:, None] * _N)
    .reshape(_ROWS // _CHUNK, _CHUNK).astype(np.int32))


@functools.partial(
    pl.kernel,
    out_type=jax.ShapeDtypeStruct((_ROWS, _D), jnp.float32),
    mesh=plsc.VectorSubcoreMesh(
        core_axis_name="c", subcore_axis_name="s",
        num_cores=_NC, num_subcores=_NS,
    ),
    scratch_types=[
        pltpu.VMEM((_NCHUNK, _CHUNK), jnp.int32),
        pltpu.VMEM((_CHUNK, _D), jnp.float32),
        pltpu.VMEM((_CHUNK, _D), jnp.float32),
        pltpu.VMEM((_CHUNK, _D), jnp.float32),
        pltpu.SemaphoreType.DMA,
        pltpu.SemaphoreType.DMA,
        pltpu.SemaphoreType.DMA,
        pltpu.SemaphoreType.DMA,
        pltpu.SemaphoreType.DMA,
        pltpu.SemaphoreType.DMA,
    ],
)
def _permute_rows(x_hbm, idx_hbm, out_hbm, idx_v, rows0, rows1, rows2,
                  gsem0, gsem1, gsem2, ssem0, ssem1, ssem2):
    wid = lax.axis_index("s") * _NC + lax.axis_index("c")
    base = wid * _ROWS_PER_W
    rows = (rows0, rows1, rows2)
    gsem = (gsem0, gsem1, gsem2)
    ssem = (ssem0, ssem1, ssem2)
    nbuf = 3
    pltpu.sync_copy(idx_hbm.at[pl.ds(wid * _NCHUNK, _NCHUNK)], idx_v)
    g = [None] * _NCHUNK
    s = [None] * _NCHUNK

    def start_load(c):
        b = c % nbuf
        g[c] = pltpu.async_copy(
            x_hbm.at[pl.ds(base + c * _CHUNK, _CHUNK)], rows[b], gsem[b])

    for c in range(nbuf - 1):
        start_load(c)
    for c in range(_NCHUNK):
        b = c % nbuf
        if c + nbuf - 1 < _NCHUNK:
            if c >= 1:
                s[c - 1].wait()
            start_load(c + nbuf - 1)
        g[c].wait()
        s[c] = pltpu.async_copy(rows[b], out_hbm.at[idx_v.at[c]], ssem[b])
    for c in range(max(0, _NCHUNK - nbuf), _NCHUNK):
        s[c].wait()


def kernel(x):
    idx = jnp.asarray(_DST)
    out = _permute_rows(x.reshape(_ROWS, _D), idx)
    return out.reshape(_B, _N, _D)


# speedup vs baseline: 1.0091x; 1.0091x over previous
import functools

import jax
import jax.numpy as jnp
import numpy as np
from jax import lax
from jax.experimental import pallas as pl
from jax.experimental.pallas import tpu as pltpu
from jax.experimental.pallas import tpu_sc as plsc

_B = 16
_N = 2048
_D = 512

_NC = 2
_NS = 16
_NW = _NC * _NS

_ROWS = _B * _N
_ROWS_PER_W = _ROWS // _NW
_CHUNK = 64
_NCHUNK = _ROWS_PER_W // _CHUNK


def _threefry2x32(k1, k2, x1, x2):
    rot = ([13, 15, 26, 6], [17, 29, 16, 24])
    with np.errstate(over="ignore"):
        ks = [k1, k2, np.uint32(k1 ^ k2 ^ np.uint32(0x1BD11BDA))]
        x = [(x1 + ks[0]).astype(np.uint32), (x2 + ks[1]).astype(np.uint32)]
        for d in range(5):
            for r in rot[d % 2]:
                x[0] = (x[0] + x[1]).astype(np.uint32)
                x[1] = ((x[1] << np.uint32(r))
                        | (x[1] >> np.uint32(32 - r))).astype(np.uint32)
                x[1] = x[0] ^ x[1]
            x[0] = (x[0] + ks[(d + 1) % 3]).astype(np.uint32)
            x[1] = (x[1] + ks[(d + 2) % 3] + np.uint32(d + 1)).astype(np.uint32)
    return x[0], x[1]


def _compute_perm(seed: int, n: int) -> np.ndarray:
    key = (np.uint32(seed >> 32), np.uint32(seed & 0xFFFFFFFF))
    x = np.arange(n)
    num_rounds = int(np.ceil(3 * np.log(max(1, n))
                             / np.log(np.iinfo(np.uint32).max)))
    for _ in range(num_rounds):
        b1, b2 = _threefry2x32(*key, np.zeros(2, np.uint32),
                               np.arange(2, dtype=np.uint32))
        key = (b1[0], b2[0])
        s1, s2 = _threefry2x32(b1[1], b2[1], np.zeros(n, np.uint32),
                               np.arange(n, dtype=np.uint32))
        x = x[np.argsort(s1 ^ s2, kind="stable")]
    return x


_PERM = _compute_perm(1234, _N)
_INV_PERM = np.argsort(_PERM)
_DST = np.ascontiguousarray(
    (_INV_PERM[None, :] + np.arange(_B, dtype=np.int64)[:, None] * _N)
    .reshape(_ROWS // _CHUNK, _CHUNK).astype(np.int32))


@functools.partial(
    pl.kernel,
    out_type=jax.ShapeDtypeStruct((_ROWS, _D), jnp.float32),
    mesh=plsc.VectorSubcoreMesh(
        core_axis_name="c", subcore_axis_name="s",
        num_cores=_NC, num_subcores=_NS,
    ),
    scratch_types=[
        pltpu.VMEM((_NCHUNK, _CHUNK), jnp.int32),
        pltpu.VMEM((_CHUNK, _D), jnp.float32),
        pltpu.VMEM((_CHUNK, _D), jnp.float32),
        pltpu.VMEM((_CHUNK, _D), jnp.float32),
        pltpu.SemaphoreType.DMA,
        pltpu.SemaphoreType.DMA,
        pltpu.SemaphoreType.DMA,
        pltpu.SemaphoreType.DMA,
        pltpu.SemaphoreType.DMA,
        pltpu.SemaphoreType.DMA,
    ],
)
def _permute_rows(x_hbm, idx_hbm, out_hbm, idx_v, rows0, rows1, rows2,
                  gsem0, gsem1, gsem2, ssem0, ssem1, ssem2):
    wid = lax.axis_index("s") * _NC + lax.axis_index("c")
    base = wid * _ROWS_PER_W
    rows = (rows0, rows1, rows2)
    gsem = (gsem0, gsem1, gsem2)
    ssem = (ssem0, ssem1, ssem2)
    nbuf = 3
    g = [None] * _NCHUNK
    s = [None] * _NCHUNK

    def start_load(c):
        b = c % nbuf
        g[c] = pltpu.async_copy(
            x_hbm.at[pl.ds(base + c * _CHUNK, _CHUNK)], rows[b], gsem[b])

    for c in range(nbuf - 1):
        start_load(c)
    pltpu.sync_copy(idx_hbm.at[pl.ds(wid * _NCHUNK, _NCHUNK)], idx_v)
    for c in range(_NCHUNK):
        b = c % nbuf
        if c + nbuf - 1 < _NCHUNK:
            if c >= 1:
                s[c - 1].wait()
            start_load(c + nbuf - 1)
        g[c].wait()
        s[c] = pltpu.async_copy(rows[b], out_hbm.at[idx_v.at[c]], ssem[b])
    for c in range(max(0, _NCHUNK - nbuf), _NCHUNK):
        s[c].wait()


def kernel(x):
    idx = jnp.asarray(_DST)
    out = _permute_rows(x.reshape(_ROWS, _D), idx)
    return out.reshape(_B, _N, _D)
